# Initial kernel scaffold; baseline (speedup 1.0000x reference)
import jax
import jax.numpy as jnp
from jax.experimental import pallas as pl


def kernel(pos, indices, W0, b0, W1, b1, W2, b2, W3, b3, W4, b4, W5, b5):
    raise NotImplementedError("write your pallas kernel here")


# speedup vs baseline: 3.2481x; 3.2481x over previous
import functools

import jax
import jax.numpy as jnp
from jax import lax
from jax.experimental import pallas as pl
from jax.experimental.pallas import tpu as pltpu
from jax.experimental.pallas import tpu_sc as plsc

_NC = 2
_NS = 16


def _elu(x):
    return jnp.where(x > 0, x, jnp.exp(jnp.minimum(x, 0.0)) - 1.0)


@functools.lru_cache(maxsize=None)
def _make_gather(B, F, CK, n_workers):
    n_chunks = B // CK
    ch_per_w = n_chunks // n_workers
    assert n_chunks % n_workers == 0 and B % CK == 0

    mesh = plsc.VectorSubcoreMesh(core_axis_name="c", subcore_axis_name="s")

    @functools.partial(
        pl.kernel,
        out_type=jax.ShapeDtypeStruct((n_chunks, CK, F), jnp.float32),
        mesh=mesh,
        scratch_types=[
            pltpu.VMEM((ch_per_w, 1, CK), jnp.int32),
            pltpu.VMEM((CK, F), jnp.float32),
            pltpu.SemaphoreType.DMA,
        ],
        compiler_params=pltpu.CompilerParams(use_tc_tiling_on_sc=False),
    )
    def gk(idx_hbm, table_hbm, out_hbm, idx_v, rows_v, sem):
        wid = lax.axis_index("s") * _NC + lax.axis_index("c")

        @pl.when(wid < n_workers)
        def _():
            c0 = wid * ch_per_w
            pltpu.sync_copy(idx_hbm.at[pl.ds(c0, ch_per_w)], idx_v)

            def body(j, carry):
                pltpu.async_copy(table_hbm.at[idx_v.at[j, 0]], rows_v, sem).wait()
                pltpu.sync_copy(rows_v, out_hbm.at[c0 + j])
                return carry

            lax.fori_loop(0, ch_per_w, body, 0)

    return gk


@functools.lru_cache(maxsize=None)
def _make_mm_elu(N, K, Fout, BN):
    def body(g_ref, w_ref, b_ref, o_ref):
        acc = jnp.dot(g_ref[...], w_ref[...], preferred_element_type=jnp.float32)
        o_ref[...] = _elu(acc + b_ref[...])

    return pl.pallas_call(
        body,
        grid=(pl.cdiv(N, BN),),
        in_specs=[
            pl.BlockSpec((BN, K), lambda i: (i, 0)),
            pl.BlockSpec((K, Fout), lambda i: (0, 0)),
            pl.BlockSpec((1, Fout), lambda i: (0, 0)),
        ],
        out_specs=pl.BlockSpec((BN, Fout), lambda i: (i, 0)),
        out_shape=jax.ShapeDtypeStruct((N, Fout), jnp.float32),
    )


@functools.lru_cache(maxsize=None)
def _make_head(N, F3, F4, C, BN):
    def body(x_ref, w4_ref, b4_ref, w5_ref, b5_ref, o_ref):
        h = _elu(
            jnp.dot(x_ref[...], w4_ref[...], preferred_element_type=jnp.float32)
            + b4_ref[...]
        )
        z = (
            jnp.dot(h, w5_ref[...], preferred_element_type=jnp.float32)
            + b5_ref[...]
        )
        m = jnp.max(z, axis=1, keepdims=True)
        lse = jnp.log(jnp.sum(jnp.exp(z - m), axis=1, keepdims=True)) + m
        o_ref[...] = z - lse

    return pl.pallas_call(
        body,
        grid=(pl.cdiv(N, BN),),
        in_specs=[
            pl.BlockSpec((BN, F3), lambda i: (i, 0)),
            pl.BlockSpec((F3, F4), lambda i: (0, 0)),
            pl.BlockSpec((1, F4), lambda i: (0, 0)),
            pl.BlockSpec((F4, C), lambda i: (0, 0)),
            pl.BlockSpec((1, C), lambda i: (0, 0)),
        ],
        out_specs=pl.BlockSpec((BN, C), lambda i: (i, 0)),
        out_shape=jax.ShapeDtypeStruct((N, C), jnp.float32),
    )


def kernel(pos, indices, W0, b0, W1, b1, W2, b2, W3, b3, W4, b4, W5, b5):
    n, s = indices.shape
    B = n * s

    CK = 125
    n_workers = 30
    idx3d = indices.reshape(B // CK, 1, CK)

    x = _make_mm_elu(n, pos.shape[1], W0.shape[1], 1024)(
        pos, W0, b0.reshape(1, -1)
    )
    for W, b in ((W1, b1), (W2, b2), (W3, b3)):
        f = x.shape[1]
        g = _make_gather(B, f, CK, n_workers)(idx3d, x)
        x = _make_mm_elu(n, s * f, W.shape[1], 1024)(
            g.reshape(n, s * f), W, b.reshape(1, -1)
        )

    return _make_head(n, W4.shape[0], W4.shape[1], W5.shape[1], 1024)(
        x, W4, b4.reshape(1, -1), W5, b5.reshape(1, -1)
    )
